# baseline (speedup 1.0000x reference)
import jax
import jax.numpy as jnp
from jax.experimental import pallas as pl
from jax.experimental.pallas import tpu as pltpu

_ROWS, _DIM = 100000, 64
_CHUNK = 1000
_NCHUNK = _ROWS // _CHUNK
_NBUF = 8


def _copy_body(x_hbm, o_hbm, *scratch):
    bufs = scratch[:_NBUF]
    sin = scratch[_NBUF:2 * _NBUF]
    sout = scratch[2 * _NBUF:]

    def start_in(g, s):
        pltpu.make_async_copy(
            x_hbm.at[pl.ds(g * _CHUNK, _CHUNK)], bufs[s], sin[s]).start()

    def wait_in(s):
        pltpu.make_async_copy(
            x_hbm.at[pl.ds(0, _CHUNK)], bufs[s], sin[s]).wait()

    def start_out(g, s):
        pltpu.make_async_copy(
            bufs[s], o_hbm.at[pl.ds(g * _CHUNK, _CHUNK)], sout[s]).start()

    def wait_out(s):
        pltpu.make_async_copy(
            bufs[s], o_hbm.at[pl.ds(0, _CHUNK)], sout[s]).wait()

    for b in range(_NBUF):
        start_in(b, b)
    for g in range(_NCHUNK):
        s = g % _NBUF
        wait_in(s)
        start_out(g, s)
        if g + _NBUF < _NCHUNK:
            wait_out(s)
            start_in(g + _NBUF, s)
    for b in range(_NBUF):
        wait_out(b)


def kernel(code_embeddings):
    return pl.pallas_call(
        _copy_body,
        out_shape=jax.ShapeDtypeStruct((_ROWS, _DIM), jnp.float32),
        in_specs=[pl.BlockSpec(memory_space=pltpu.MemorySpace.HBM)],
        out_specs=pl.BlockSpec(memory_space=pltpu.MemorySpace.HBM),
        scratch_shapes=(
            [pltpu.VMEM((_CHUNK, _DIM), jnp.float32)] * _NBUF
            + [pltpu.SemaphoreType.DMA] * (2 * _NBUF)
        ),
    )(code_embeddings)


# speedup vs baseline: 1.1902x; 1.1902x over previous
import jax
import jax.numpy as jnp
from jax import lax
from jax.experimental import pallas as pl
from jax.experimental.pallas import tpu as pltpu
from jax.experimental.pallas import tpu_sc as plsc

_ROWS, _DIM = 100000, 64
_CHUNK = 200
_NCHUNK = _ROWS // _CHUNK
_NW = 16
_G = _NCHUNK // _NW
_R = _NCHUNK % _NW
_NBUF = 4


def _copy_body(x_hbm, o_hbm, *scratch):
    bufs = scratch[:_NBUF]
    sin = scratch[_NBUF:2 * _NBUF]
    sout = scratch[2 * _NBUF:]
    wid = lax.axis_index("s")

    def start_in(g, s):
        base = (wid + g * _NW) * _CHUNK
        pltpu.make_async_copy(x_hbm.at[pl.ds(base, _CHUNK)], bufs[s], sin[s]).start()

    def wait_in(s):
        pltpu.make_async_copy(x_hbm.at[pl.ds(0, _CHUNK)], bufs[s], sin[s]).wait()

    def start_out(g, s):
        base = (wid + g * _NW) * _CHUNK
        pltpu.make_async_copy(bufs[s], o_hbm.at[pl.ds(base, _CHUNK)], sout[s]).start()

    def wait_out(s):
        pltpu.make_async_copy(bufs[s], o_hbm.at[pl.ds(0, _CHUNK)], sout[s]).wait()

    for b in range(_NBUF):
        start_in(b, b)

    for g in range(_G):
        s = g % _NBUF
        wait_in(s)
        start_out(g, s)
        if g + _NBUF < _G:
            wait_out(s)
            start_in(g + _NBUF, s)
        elif g + _NBUF == _G:
            @pl.when(wid < _R)
            def _():
                wait_out(s)
                start_in(_G, s)

    @pl.when(wid < _R)
    def _():
        s = _G % _NBUF
        wait_in(s)
        start_out(_G, s)

    for b in range(_NBUF):
        wait_out(b)


def kernel(code_embeddings):
    k = pl.kernel(
        _copy_body,
        out_type=jax.ShapeDtypeStruct((_ROWS, _DIM), jnp.float32),
        mesh=plsc.VectorSubcoreMesh(
            core_axis_name="c", subcore_axis_name="s", num_cores=1),
        scratch_types=(
            [pltpu.VMEM((_CHUNK, _DIM), jnp.float32)] * _NBUF
            + [pltpu.SemaphoreType.DMA] * (2 * _NBUF)
        ),
    )
    return k(code_embeddings)


# speedup vs baseline: 1.2452x; 1.0462x over previous
import jax
import jax.numpy as jnp
from jax import lax
from jax.experimental import pallas as pl
from jax.experimental.pallas import tpu as pltpu
from jax.experimental.pallas import tpu_sc as plsc

_ROWS, _DIM = 100000, 64
_CHUNK = 400
_NCHUNK = _ROWS // _CHUNK
_NW = 32
_G = _NCHUNK // _NW
_R = _NCHUNK % _NW
_NBUF = 2


def _copy_body(x_hbm, o_hbm, *scratch):
    bufs = scratch[:_NBUF]
    sin = scratch[_NBUF:2 * _NBUF]
    sout = scratch[2 * _NBUF:]
    wid = lax.axis_index("c") * 16 + lax.axis_index("s")

    def start_in(g, s):
        base = (wid + g * _NW) * _CHUNK
        pltpu.make_async_copy(x_hbm.at[pl.ds(base, _CHUNK)], bufs[s], sin[s]).start()

    def wait_in(s):
        pltpu.make_async_copy(x_hbm.at[pl.ds(0, _CHUNK)], bufs[s], sin[s]).wait()

    def start_out(g, s):
        base = (wid + g * _NW) * _CHUNK
        pltpu.make_async_copy(bufs[s], o_hbm.at[pl.ds(base, _CHUNK)], sout[s]).start()

    def wait_out(s):
        pltpu.make_async_copy(bufs[s], o_hbm.at[pl.ds(0, _CHUNK)], sout[s]).wait()

    for b in range(_NBUF):
        start_in(b, b)

    for g in range(_G):
        s = g % _NBUF
        wait_in(s)
        start_out(g, s)
        if g + _NBUF < _G:
            wait_out(s)
            start_in(g + _NBUF, s)
        elif g + _NBUF == _G:
            @pl.when(wid < _R)
            def _():
                wait_out(s)
                start_in(_G, s)

    @pl.when(wid < _R)
    def _():
        s = _G % _NBUF
        wait_in(s)
        start_out(_G, s)

    for b in range(_NBUF):
        wait_out(b)


def kernel(code_embeddings):
    k = pl.kernel(
        _copy_body,
        out_type=jax.ShapeDtypeStruct((_ROWS, _DIM), jnp.float32),
        mesh=plsc.VectorSubcoreMesh(core_axis_name="c", subcore_axis_name="s"),
        scratch_types=(
            [pltpu.VMEM((_CHUNK, _DIM), jnp.float32)] * _NBUF
            + [pltpu.SemaphoreType.DMA] * (2 * _NBUF)
        ),
    )
    return k(code_embeddings)
